# baseline (speedup 1.0000x reference)
import functools

import jax
import jax.numpy as jnp
from jax import lax
from jax.experimental import pallas as pl
from jax.experimental.pallas import tpu as pltpu
from jax.experimental.pallas import tpu_sc as plsc

V = 100000
H = 768
DK = 64

NW = 32


def _sc_gather_word(ids3, word_emb):
    nw, nch, ch = ids3.shape
    nt = nw * nch * ch
    tpw = nch * ch

    mesh = plsc.VectorSubcoreMesh(core_axis_name="c", subcore_axis_name="s")

    @functools.partial(
        pl.kernel,
        mesh=mesh,
        out_type=jax.ShapeDtypeStruct((nt, H), jnp.float32),
        scratch_types=[
            pltpu.VMEM((nch, ch), jnp.int32),
            pltpu.VMEM((ch, H), jnp.float32),
            pltpu.VMEM((ch, H), jnp.float32),
            pltpu.SemaphoreType.DMA,
            pltpu.SemaphoreType.DMA,
        ],
    )
    def gather_kernel(ids_hbm, wtab_hbm, oute_hbm, idx_v, buf0, buf1, sem0, sem1):
        wid = lax.axis_index("s") * 2 + lax.axis_index("c")
        base = wid * tpw
        pltpu.sync_copy(ids_hbm.at[wid], idx_v)
        bufs = (buf0, buf1)
        sems = (sem0, sem1)
        copies = [None, None]
        copies[0] = pltpu.async_copy(wtab_hbm.at[idx_v.at[0]], buf0, sem0)
        for j in range(nch):
            if j + 1 < nch:
                copies[(j + 1) % 2] = pltpu.async_copy(
                    wtab_hbm.at[idx_v.at[j + 1]], bufs[(j + 1) % 2], sems[(j + 1) % 2])
            copies[j % 2].wait()
            pltpu.sync_copy(bufs[j % 2], oute_hbm.at[pl.ds(base + j * ch, ch)])

    return gather_kernel(ids3, word_emb)


OFFP = 50176
PBLK = 512


def _pair_body(ptA_ref, ptB_ref, out_ref):
    both = jnp.concatenate([ptA_ref[...], ptB_ref[...]], axis=0)
    out_ref[...] = both.T


def _build_pairs(pt):
    grid = OFFP // PBLK
    return pl.pallas_call(
        _pair_body,
        grid=(grid,),
        in_specs=[
            pl.BlockSpec((DK, PBLK), lambda i: (0, i)),
            pl.BlockSpec((DK, PBLK), lambda i: (0, grid + i)),
        ],
        out_specs=pl.BlockSpec((PBLK, 2 * DK), lambda i: (i, 0)),
        out_shape=jax.ShapeDtypeStruct((OFFP, 2 * DK), jnp.float32),
        compiler_params=pltpu.CompilerParams(
            dimension_semantics=("arbitrary",),
        ),
    )(pt, pt)


def _sc_gather_prom(pids3, ppair):
    nw, nch, ch = pids3.shape
    nt = nw * nch * ch
    tpw = nch * ch
    dp = ppair.shape[1]

    mesh = plsc.VectorSubcoreMesh(core_axis_name="c", subcore_axis_name="s")

    @functools.partial(
        pl.kernel,
        mesh=mesh,
        out_type=jax.ShapeDtypeStruct((nt, dp), jnp.float32),
        scratch_types=[
            pltpu.VMEM((nch, ch), jnp.int32),
            pltpu.VMEM((tpw, dp), jnp.float32),
            pltpu.SemaphoreType.DMA,
        ],
    )
    def gather_kernel(pids_hbm, ptab_hbm, outp_hbm, pidx_v, pbuf, semp):
        wid = lax.axis_index("s") * 2 + lax.axis_index("c")
        base = wid * tpw
        pltpu.sync_copy(pids_hbm.at[wid], pidx_v)
        pcopies = []
        for j in range(nch):
            pcopies.append(pltpu.async_copy(
                ptab_hbm.at[pidx_v.at[j]], pbuf.at[pl.ds(j * ch, ch)], semp))
        for c in pcopies:
            c.wait()
        pltpu.sync_copy(pbuf, outp_hbm.at[pl.ds(base, tpw)])

    return gather_kernel(pids3, ppair)


def _tc_body(vals_ref, half_ref, e_ref, p_ref, pos_ref, tt_ref, promw_ref,
             promb_ref, fcw0_ref, fcw1_ref, fcw2_ref, fcb_ref, ln1g_ref,
             ln1b_ref, projw_ref, projb_ref, ln2g_ref, ln2b_ref, out_ref):
    e = e_ref[...]
    p2 = p_ref[...]
    half = half_ref[0, 0, :][:, None]
    p_lo = p2[:, :DK]
    p_hi = p2[:, DK:]
    p = jnp.where(half > 0.5, p_hi, p_lo)
    pe = jnp.dot(p, promw_ref[...], preferred_element_type=jnp.float32)
    pe = pe + promb_ref[...]
    h = jnp.dot(e, fcw0_ref[...], preferred_element_type=jnp.float32)
    h = h + jnp.dot(pe, fcw1_ref[...], preferred_element_type=jnp.float32)
    vals = vals_ref[0, 0, :]
    h = h + vals[:, None] * fcw2_ref[...]
    h = h + fcb_ref[...]
    m = jnp.mean(h, axis=-1, keepdims=True)
    c = h - m
    var = jnp.mean(c * c, axis=-1, keepdims=True)
    h = c * lax.rsqrt(var + 1e-12) * ln1g_ref[...] + ln1b_ref[...]
    h = h * jax.nn.sigmoid(1.702 * h)
    o = jnp.dot(h, projw_ref[...], preferred_element_type=jnp.float32)
    o = o + projb_ref[...] + tt_ref[0:1, :] + pos_ref[...]
    m2 = jnp.mean(o, axis=-1, keepdims=True)
    c2 = o - m2
    var2 = jnp.mean(c2 * c2, axis=-1, keepdims=True)
    out_ref[...] = c2 * lax.rsqrt(var2 + 1e-12) * ln2g_ref[...] + ln2b_ref[...]


def _tc_dense(vals3, half3, e, p2, pos_emb, tt_emb, prom_w, prom_b, fc_w0,
              fc_w1, fc_w2, fc_b, ln1_g, ln1_b, proj_w, proj_b, ln2_g, ln2_b,
              tb, seq, interpret=False):
    nt = e.shape[0]
    grid = nt // tb
    pos_blocks = seq // tb if seq >= tb else 1

    full = lambda shape: pl.BlockSpec(shape, lambda i: (0, 0))
    return pl.pallas_call(
        _tc_body,
        grid=(grid,),
        in_specs=[
            pl.BlockSpec((1, 1, tb), lambda i: (i, 0, 0)),
            pl.BlockSpec((1, 1, tb), lambda i: (i, 0, 0)),
            pl.BlockSpec((tb, H), lambda i: (i, 0)),
            pl.BlockSpec((tb, 2 * DK), lambda i: (i, 0)),
            pl.BlockSpec((tb, H), lambda i: (i % pos_blocks, 0)),
            full(tt_emb.shape),
            full((DK, H)),
            full((1, H)),
            full((H, H)),
            full((H, H)),
            full((1, H)),
            full((1, H)),
            full((1, H)),
            full((1, H)),
            full((H, H)),
            full((1, H)),
            full((1, H)),
            full((1, H)),
        ],
        out_specs=pl.BlockSpec((tb, H), lambda i: (i, 0)),
        out_shape=jax.ShapeDtypeStruct((nt, H), jnp.float32),
        compiler_params=pltpu.CompilerParams(
            dimension_semantics=("arbitrary",),
        ),
        interpret=interpret,
    )(vals3, half3, e, p2, pos_emb, tt_emb, prom_w, prom_b, fc_w0, fc_w1,
      fc_w2, fc_b, ln1_g, ln1_b, proj_w, proj_b, ln2_g, ln2_b)


def kernel(input_ids, values, word_emb, pos_emb, tt_emb, promoter_knowledge,
           prom_w, prom_b, fc_w, fc_b, ln1_g, ln1_b, proj_w, proj_b,
           ln2_g, ln2_b):
    b, s = input_ids.shape
    nt = b * s
    ids = input_ids.reshape(-1).astype(jnp.int32)
    ch = 64
    nch = nt // (NW * ch)
    ids3 = ids.reshape(NW, nch, ch)
    pslot3 = jnp.where(ids < OFFP, ids, ids - OFFP).reshape(NW, nch, ch)

    e = _sc_gather_word(ids3, word_emb)
    ppair = _build_pairs(promoter_knowledge.T)
    p2 = _sc_gather_prom(pslot3, ppair)

    tb = 512
    vals3 = values.reshape(nt // tb, 1, tb)
    half3 = (ids >= OFFP).astype(jnp.float32).reshape(nt // tb, 1, tb)
    row = lambda v: v.reshape(1, H)
    out = _tc_dense(
        vals3, half3, e, p2, pos_emb, tt_emb, prom_w, row(prom_b),
        fc_w[:H], fc_w[H:2 * H], fc_w[2 * H:2 * H + 1], row(fc_b),
        row(ln1_g), row(ln1_b), proj_w, row(proj_b), row(ln2_g), row(ln2_b),
        tb, s)
    return out.reshape(b, s, H)


# speedup vs baseline: 1.2561x; 1.2561x over previous
import functools

import jax
import jax.numpy as jnp
from jax import lax
from jax.experimental import pallas as pl
from jax.experimental.pallas import tpu as pltpu
from jax.experimental.pallas import tpu_sc as plsc

V = 100000
H = 768
DK = 64

NW = 32


def _sc_gather_both(ids3, pids3, word_emb, ppair):
    nw, nch, ch = ids3.shape
    nt = nw * nch * ch
    tpw = nch * ch
    dp = ppair.shape[1]

    mesh = plsc.VectorSubcoreMesh(core_axis_name="c", subcore_axis_name="s")

    _, nchp, chp = pids3.shape

    @functools.partial(
        pl.kernel,
        mesh=mesh,
        out_type=(
            jax.ShapeDtypeStruct((nt, H), jnp.float32),
            jax.ShapeDtypeStruct((nt, dp), jnp.float32),
        ),
        scratch_types=[
            pltpu.VMEM((nch, ch), jnp.int32),
            pltpu.VMEM((nchp, chp), jnp.int32),
            pltpu.VMEM((ch, H), jnp.float32),
            pltpu.VMEM((ch, H), jnp.float32),
            pltpu.VMEM((tpw, dp), jnp.float32),
            pltpu.SemaphoreType.DMA,
            pltpu.SemaphoreType.DMA,
            pltpu.SemaphoreType.DMA,
        ],
    )
    def gather_kernel(ids_hbm, pids_hbm, wtab_hbm, ptab_hbm, oute_hbm, outp_hbm,
                      idx_v, pidx_v, buf0, buf1, pbuf, sem0, sem1, semp):
        wid = lax.axis_index("s") * 2 + lax.axis_index("c")
        base = wid * tpw
        pltpu.sync_copy(ids_hbm.at[wid], idx_v)
        pltpu.sync_copy(pids_hbm.at[wid], pidx_v)
        pcopies = []
        for j in range(nchp):
            pcopies.append(pltpu.async_copy(
                ptab_hbm.at[pidx_v.at[j]], pbuf.at[pl.ds(j * chp, chp)], semp))
        bufs = (buf0, buf1)
        sems = (sem0, sem1)
        copies = [None, None]
        copies[0] = pltpu.async_copy(wtab_hbm.at[idx_v.at[0]], buf0, sem0)
        for j in range(nch):
            if j + 1 < nch:
                copies[(j + 1) % 2] = pltpu.async_copy(
                    wtab_hbm.at[idx_v.at[j + 1]], bufs[(j + 1) % 2], sems[(j + 1) % 2])
            copies[j % 2].wait()
            pltpu.sync_copy(bufs[j % 2], oute_hbm.at[pl.ds(base + j * ch, ch)])
        for c in pcopies:
            c.wait()
        pltpu.sync_copy(pbuf, outp_hbm.at[pl.ds(base, tpw)])

    return gather_kernel(ids3, pids3, word_emb, ppair)


OFFP = 51200
PBLK = 2048


def _pair_body(ptA_ref, ptB_ref, out_ref):
    both = jnp.concatenate([ptA_ref[...], ptB_ref[...]], axis=0)
    out_ref[...] = both.T


def _build_pairs(pt):
    grid = OFFP // PBLK
    return pl.pallas_call(
        _pair_body,
        grid=(grid,),
        in_specs=[
            pl.BlockSpec((DK, PBLK), lambda i: (0, i)),
            pl.BlockSpec((DK, PBLK),
                         lambda i: (0, jnp.minimum(grid + i, 2 * grid - 2))),
        ],
        out_specs=pl.BlockSpec((PBLK, 2 * DK), lambda i: (i, 0)),
        out_shape=jax.ShapeDtypeStruct((OFFP, 2 * DK), jnp.float32),
        compiler_params=pltpu.CompilerParams(
            dimension_semantics=("arbitrary",),
        ),
    )(pt, pt)


def _tc_body(vals_ref, half_ref, e_ref, p_ref, pos_ref, tt_ref, promw_ref,
             promb_ref, fcw0_ref, fcw1_ref, fcw2_ref, fcb_ref, ln1g_ref,
             ln1b_ref, projw_ref, projb_ref, ln2g_ref, ln2b_ref, out_ref):
    e = e_ref[...]
    p2 = p_ref[...]
    half = half_ref[0, 0, :][:, None]
    p_lo = p2[:, :DK]
    p_hi = p2[:, DK:]
    p = jnp.where(half > 0.5, p_hi, p_lo)
    b16 = lambda x: x.astype(jnp.bfloat16)
    pe = jnp.dot(b16(p), b16(promw_ref[...]), preferred_element_type=jnp.float32)
    pe = pe + promb_ref[...]
    h = jnp.dot(b16(e), b16(fcw0_ref[...]), preferred_element_type=jnp.float32)
    h = h + jnp.dot(b16(pe), b16(fcw1_ref[...]), preferred_element_type=jnp.float32)
    vals = vals_ref[0, 0, :]
    h = h + vals[:, None] * fcw2_ref[...]
    h = h + fcb_ref[...]
    m = jnp.mean(h, axis=-1, keepdims=True)
    c = h - m
    var = jnp.mean(c * c, axis=-1, keepdims=True)
    h = c * lax.rsqrt(var + 1e-12) * ln1g_ref[...] + ln1b_ref[...]
    h = h * jax.nn.sigmoid(1.702 * h)
    o = jnp.dot(b16(h), b16(projw_ref[...]), preferred_element_type=jnp.float32)
    o = o + projb_ref[...] + tt_ref[0:1, :] + pos_ref[...]
    m2 = jnp.mean(o, axis=-1, keepdims=True)
    c2 = o - m2
    var2 = jnp.mean(c2 * c2, axis=-1, keepdims=True)
    out_ref[...] = c2 * lax.rsqrt(var2 + 1e-12) * ln2g_ref[...] + ln2b_ref[...]


def _tc_dense(vals3, half3, e, p2, pos_emb, tt_emb, prom_w, prom_b, fc_w0,
              fc_w1, fc_w2, fc_b, ln1_g, ln1_b, proj_w, proj_b, ln2_g, ln2_b,
              tb, seq, interpret=False):
    nt = e.shape[0]
    grid = nt // tb
    pos_blocks = seq // tb if seq >= tb else 1

    full = lambda shape: pl.BlockSpec(shape, lambda i: (0, 0))
    return pl.pallas_call(
        _tc_body,
        grid=(grid,),
        in_specs=[
            pl.BlockSpec((1, 1, tb), lambda i: (i, 0, 0)),
            pl.BlockSpec((1, 1, tb), lambda i: (i, 0, 0)),
            pl.BlockSpec((tb, H), lambda i: (i, 0)),
            pl.BlockSpec((tb, 2 * DK), lambda i: (i, 0)),
            pl.BlockSpec((tb, H), lambda i: (i % pos_blocks, 0)),
            full(tt_emb.shape),
            full((DK, H)),
            full((1, H)),
            full((H, H)),
            full((H, H)),
            full((1, H)),
            full((1, H)),
            full((1, H)),
            full((1, H)),
            full((H, H)),
            full((1, H)),
            full((1, H)),
            full((1, H)),
        ],
        out_specs=pl.BlockSpec((tb, H), lambda i: (i, 0)),
        out_shape=jax.ShapeDtypeStruct((nt, H), jnp.float32),
        compiler_params=pltpu.CompilerParams(
            dimension_semantics=("arbitrary",),
        ),
        interpret=interpret,
    )(vals3, half3, e, p2, pos_emb, tt_emb, prom_w, prom_b, fc_w0, fc_w1,
      fc_w2, fc_b, ln1_g, ln1_b, proj_w, proj_b, ln2_g, ln2_b)


def kernel(input_ids, values, word_emb, pos_emb, tt_emb, promoter_knowledge,
           prom_w, prom_b, fc_w, fc_b, ln1_g, ln1_b, proj_w, proj_b,
           ln2_g, ln2_b):
    b, s = input_ids.shape
    nt = b * s
    ids = input_ids.reshape(-1).astype(jnp.int32)
    ch = 32
    nch = nt // (NW * ch)
    ids3 = ids.reshape(NW, nch, ch)
    chp = 64
    nchp = nt // (NW * chp)
    pslot3 = jnp.where(ids < OFFP, ids, ids - OFFP).reshape(NW, nchp, chp)

    ppair = _build_pairs(promoter_knowledge.T)
    e, p2 = _sc_gather_both(ids3, pslot3, word_emb, ppair)

    tb = 512
    vals3 = values.reshape(nt // tb, 1, tb)
    half3 = (ids >= OFFP).astype(jnp.float32).reshape(nt // tb, 1, tb)
    row = lambda v: v.reshape(1, H)
    out = _tc_dense(
        vals3, half3, e, p2, pos_emb, tt_emb, prom_w, row(prom_b),
        fc_w[:H], fc_w[H:2 * H], fc_w[2 * H:2 * H + 1], row(fc_b),
        row(ln1_g), row(ln1_b), proj_w, row(proj_b), row(ln2_g), row(ln2_b),
        tb, s)
    return out.reshape(b, s, H)


# speedup vs baseline: 1.2748x; 1.0149x over previous
import functools

import jax
import jax.numpy as jnp
from jax import lax
from jax.experimental import pallas as pl
from jax.experimental.pallas import tpu as pltpu
from jax.experimental.pallas import tpu_sc as plsc

V = 100000
H = 768
DK = 64

NW = 32


def _sc_gather_both(ids3, pids3, word_emb, ppair):
    nw, nch, ch = ids3.shape
    nt = nw * nch * ch
    tpw = nch * ch
    dp = ppair.shape[1]

    mesh = plsc.VectorSubcoreMesh(core_axis_name="c", subcore_axis_name="s")

    _, nchp, chp = pids3.shape

    @functools.partial(
        pl.kernel,
        mesh=mesh,
        out_type=(
            jax.ShapeDtypeStruct((nt, H), jnp.float32),
            jax.ShapeDtypeStruct((nt, dp), jnp.float32),
        ),
        scratch_types=[
            pltpu.VMEM((nch, ch), jnp.int32),
            pltpu.VMEM((nchp, chp), jnp.int32),
            pltpu.VMEM((ch, H), jnp.float32),
            pltpu.VMEM((ch, H), jnp.float32),
            pltpu.VMEM((tpw, dp), jnp.float32),
            pltpu.SemaphoreType.DMA,
            pltpu.SemaphoreType.DMA,
            pltpu.SemaphoreType.DMA,
        ],
    )
    def gather_kernel(ids_hbm, pids_hbm, wtab_hbm, ptab_hbm, oute_hbm, outp_hbm,
                      idx_v, pidx_v, buf0, buf1, pbuf, sem0, sem1, semp):
        wid = lax.axis_index("s") * 2 + lax.axis_index("c")
        base = wid * tpw
        pltpu.sync_copy(ids_hbm.at[wid], idx_v)
        pltpu.sync_copy(pids_hbm.at[wid], pidx_v)
        pcopies = []
        for j in range(nchp):
            pcopies.append(pltpu.async_copy(
                ptab_hbm.at[pidx_v.at[j]], pbuf.at[pl.ds(j * chp, chp)], semp))
        bufs = (buf0, buf1)
        sems = (sem0, sem1)
        copies = [None, None]
        copies[0] = pltpu.async_copy(wtab_hbm.at[idx_v.at[0]], buf0, sem0)
        for j in range(nch):
            if j + 1 < nch:
                copies[(j + 1) % 2] = pltpu.async_copy(
                    wtab_hbm.at[idx_v.at[j + 1]], bufs[(j + 1) % 2], sems[(j + 1) % 2])
            copies[j % 2].wait()
            pltpu.sync_copy(bufs[j % 2], oute_hbm.at[pl.ds(base + j * ch, ch)])
        for c in pcopies:
            c.wait()
        pltpu.sync_copy(pbuf, outp_hbm.at[pl.ds(base, tpw)])

    return gather_kernel(ids3, pids3, word_emb, ppair)


OFFP = 51200
PBLK = 2048


def _pair_body(ptA_ref, ptB_ref, out_ref):
    both = jnp.concatenate([ptA_ref[...], ptB_ref[...]], axis=0)
    out_ref[...] = both.T


def _build_pairs(pt):
    grid = OFFP // PBLK
    return pl.pallas_call(
        _pair_body,
        grid=(grid,),
        in_specs=[
            pl.BlockSpec((DK, PBLK), lambda i: (0, i)),
            pl.BlockSpec((DK, PBLK),
                         lambda i: (0, jnp.minimum(grid + i, 2 * grid - 2))),
        ],
        out_specs=pl.BlockSpec((PBLK, 2 * DK), lambda i: (i, 0)),
        out_shape=jax.ShapeDtypeStruct((OFFP, 2 * DK), jnp.float32),
        compiler_params=pltpu.CompilerParams(
            dimension_semantics=("arbitrary",),
        ),
    )(pt, pt)


def _tc_body(vals_ref, half_ref, e_ref, p_ref, pos_ref, tt_ref, promw_ref,
             promb_ref, fcw0_ref, fcw1_ref, fcw2_ref, fcb_ref, ln1g_ref,
             ln1b_ref, projw_ref, projb_ref, ln2g_ref, ln2b_ref, out_ref):
    e = e_ref[...]
    p2 = p_ref[...]
    half = half_ref[0, 0, :][:, None]
    p_lo = p2[:, :DK]
    p_hi = p2[:, DK:]
    p = jnp.where(half > 0.5, p_hi, p_lo)
    b16 = lambda x: x.astype(jnp.bfloat16)
    pe = jnp.dot(b16(p), b16(promw_ref[...]), preferred_element_type=jnp.float32)
    pe = pe + promb_ref[...]
    h = jnp.dot(b16(e), b16(fcw0_ref[...]), preferred_element_type=jnp.float32)
    h = h + jnp.dot(b16(pe), b16(fcw1_ref[...]), preferred_element_type=jnp.float32)
    vals = vals_ref[0, 0, :]
    h = h + vals[:, None] * fcw2_ref[...]
    h = h + fcb_ref[...]
    m = jnp.mean(h, axis=-1, keepdims=True)
    c = h - m
    var = jnp.mean(c * c, axis=-1, keepdims=True)
    h = c * lax.rsqrt(var + 1e-12) * ln1g_ref[...] + ln1b_ref[...]
    h = h * jax.nn.sigmoid(1.702 * h)
    o = jnp.dot(b16(h), b16(projw_ref[...]), preferred_element_type=jnp.float32)
    o = o + projb_ref[...] + tt_ref[0:1, :] + pos_ref[...]
    m2 = jnp.mean(o, axis=-1, keepdims=True)
    c2 = o - m2
    var2 = jnp.mean(c2 * c2, axis=-1, keepdims=True)
    out_ref[...] = c2 * lax.rsqrt(var2 + 1e-12) * ln2g_ref[...] + ln2b_ref[...]


def _tc_dense(vals3, half3, e, p2, pos_emb, tt_emb, prom_w, prom_b, fc_w0,
              fc_w1, fc_w2, fc_b, ln1_g, ln1_b, proj_w, proj_b, ln2_g, ln2_b,
              tb, seq, interpret=False):
    nt = e.shape[0]
    grid = nt // tb
    pos_blocks = seq // tb if seq >= tb else 1

    full = lambda shape: pl.BlockSpec(shape, lambda i: (0, 0))
    return pl.pallas_call(
        _tc_body,
        grid=(grid,),
        in_specs=[
            pl.BlockSpec((1, 1, tb), lambda i: (i, 0, 0)),
            pl.BlockSpec((1, 1, tb), lambda i: (i, 0, 0)),
            pl.BlockSpec((tb, H), lambda i: (i, 0)),
            pl.BlockSpec((tb, 2 * DK), lambda i: (i, 0)),
            pl.BlockSpec((tb, H), lambda i: (i % pos_blocks, 0)),
            full(tt_emb.shape),
            full((DK, H)),
            full((1, H)),
            full((H, H)),
            full((H, H)),
            full((1, H)),
            full((1, H)),
            full((1, H)),
            full((1, H)),
            full((H, H)),
            full((1, H)),
            full((1, H)),
            full((1, H)),
        ],
        out_specs=pl.BlockSpec((tb, H), lambda i: (i, 0)),
        out_shape=jax.ShapeDtypeStruct((nt, H), jnp.float32),
        compiler_params=pltpu.CompilerParams(
            dimension_semantics=("arbitrary",),
        ),
        interpret=interpret,
    )(vals3, half3, e, p2, pos_emb, tt_emb, prom_w, prom_b, fc_w0, fc_w1,
      fc_w2, fc_b, ln1_g, ln1_b, proj_w, proj_b, ln2_g, ln2_b)


def kernel(input_ids, values, word_emb, pos_emb, tt_emb, promoter_knowledge,
           prom_w, prom_b, fc_w, fc_b, ln1_g, ln1_b, proj_w, proj_b,
           ln2_g, ln2_b):
    b, s = input_ids.shape
    nt = b * s
    ids = input_ids.reshape(-1).astype(jnp.int32)
    ch = 32
    nch = nt // (NW * ch)
    ids3 = ids.reshape(NW, nch, ch)
    chp = 64
    nchp = nt // (NW * chp)
    pslot3 = jnp.where(ids < OFFP, ids, ids - OFFP).reshape(NW, nchp, chp)

    ppair = _build_pairs(promoter_knowledge.T)
    e, p2 = _sc_gather_both(ids3, pslot3, word_emb, ppair)

    tb = 1024
    vals3 = values.reshape(nt // tb, 1, tb)
    half3 = (ids >= OFFP).astype(jnp.float32).reshape(nt // tb, 1, tb)
    row = lambda v: v.reshape(1, H)
    out = _tc_dense(
        vals3, half3, e, p2, pos_emb, tt_emb, prom_w, row(prom_b),
        fc_w[:H], fc_w[H:2 * H], fc_w[2 * H:2 * H + 1], row(fc_b),
        row(ln1_g), row(ln1_b), proj_w, row(proj_b), row(ln2_g), row(ln2_b),
        tb, s)
    return out.reshape(b, s, H)
